# Initial kernel scaffold; baseline (speedup 1.0000x reference)
import jax
import jax.numpy as jnp
from jax.experimental import pallas as pl


def kernel(x, edge_index, Wq1, bq1, Wk1, bk1, Wv1, bv1, Ws1, bs1, Wq2, bq2, Wk2, bk2, Wv2, bv2, Ws2, bs2):
    raise NotImplementedError("write your pallas kernel here")


# speedup vs baseline: 8.8287x; 8.8287x over previous
import math

import jax
import jax.numpy as jnp
from jax import lax
from jax.experimental import pallas as pl
from jax.experimental.pallas import tpu as pltpu
from jax.experimental.pallas import tpu_sc as plsc

_C = 64
_NSC = 2
_NTEC = 16
_NW = _NSC * _NTEC
_LANES = 16


def _proj4(xp, Wq, bq, Wk, bk, Wv, bv, Ws, bs):
    NP, D = xp.shape
    R = NP // 4
    bspec_x = pl.BlockSpec((R, D), lambda i: (i, 0))
    bspec_w = pl.BlockSpec((D, D), lambda i: (0, 0))
    bspec_b = pl.BlockSpec((1, D), lambda i: (0, 0))

    def body(x_ref, wq, bq_r, wk, bk_r, wv, bv_r, ws, bs_r, qo, ko, vo, so):
        xb = x_ref[...]
        qo[...] = jnp.dot(xb, wq[...], preferred_element_type=jnp.float32) + bq_r[...]
        ko[...] = jnp.dot(xb, wk[...], preferred_element_type=jnp.float32) + bk_r[...]
        vo[...] = jnp.dot(xb, wv[...], preferred_element_type=jnp.float32) + bv_r[...]
        so[...] = jnp.dot(xb, ws[...], preferred_element_type=jnp.float32) + bs_r[...]

    outs = [jax.ShapeDtypeStruct((NP, D), jnp.float32)] * 4
    return pl.pallas_call(
        body,
        grid=(4,),
        in_specs=[bspec_x] + [bspec_w, bspec_b] * 4,
        out_specs=[bspec_x] * 4,
        out_shape=outs,
    )(xp, Wq, bq.reshape(1, D), Wk, bk.reshape(1, D),
      Wv, bv.reshape(1, D), Ws, bs.reshape(1, D))


def _combine_proj4(num_p, den_p, skip, Wq, bq, Wk, bk, Wv, bv, Ws, bs):
    _, NP, D = num_p.shape
    R = NP // 4
    bspec_n = pl.BlockSpec((_NSC, R, D), lambda i: (0, i, 0))
    bspec_d = pl.BlockSpec((R, _NW), lambda i: (i, 0))
    bspec_x = pl.BlockSpec((R, D), lambda i: (i, 0))
    bspec_w = pl.BlockSpec((D, D), lambda i: (0, 0))
    bspec_b = pl.BlockSpec((1, D), lambda i: (0, 0))

    def body(num_ref, den_ref, skip_ref, wq, bq_r, wk, bk_r, wv, bv_r, ws, bs_r,
             qo, ko, vo, so):
        num = num_ref[0] + num_ref[1]
        den = jnp.sum(den_ref[...], axis=1, keepdims=True)
        h = num / (den + 1e-16) + skip_ref[...]
        h = jnp.where(h >= 0, h, 0.1 * h)
        qo[...] = jnp.dot(h, wq[...], preferred_element_type=jnp.float32) + bq_r[...]
        ko[...] = jnp.dot(h, wk[...], preferred_element_type=jnp.float32) + bk_r[...]
        vo[...] = jnp.dot(h, wv[...], preferred_element_type=jnp.float32) + bv_r[...]
        so[...] = jnp.dot(h, ws[...], preferred_element_type=jnp.float32) + bs_r[...]

    outs = [jax.ShapeDtypeStruct((NP, D), jnp.float32)] * 4
    return pl.pallas_call(
        body,
        grid=(4,),
        in_specs=[bspec_n, bspec_d, bspec_x] + [bspec_w, bspec_b] * 4,
        out_specs=[bspec_x] * 4,
        out_shape=outs,
    )(num_p, den_p, skip, Wq, bq.reshape(1, D), Wk, bk.reshape(1, D),
      Wv, bv.reshape(1, D), Ws, bs.reshape(1, D))


def _combine_final(num_p, den_p, skip):
    _, NP, D = num_p.shape
    R = NP // 4
    bspec_n = pl.BlockSpec((_NSC, R, D), lambda i: (0, i, 0))
    bspec_d = pl.BlockSpec((R, _NW), lambda i: (i, 0))
    bspec_x = pl.BlockSpec((R, D), lambda i: (i, 0))

    def body(num_ref, den_ref, skip_ref, out_ref):
        num = num_ref[0] + num_ref[1]
        den = jnp.sum(den_ref[...], axis=1, keepdims=True)
        out_ref[...] = num / (den + 1e-16) + skip_ref[...]

    return pl.pallas_call(
        body,
        grid=(4,),
        in_specs=[bspec_n, bspec_d, bspec_x],
        out_specs=bspec_x,
        out_shape=jax.ShapeDtypeStruct((NP, D), jnp.float32),
    )(num_p, den_p, skip)


def _edge_pass(q, k, v, srcp, dstp, zeros_n, zeros_d):
    NP, D = q.shape
    EP = srcp.shape[0]
    EPW = EP // _NW
    T = EPW // _C
    RT = NP // _NTEC
    ND = D // _LANES
    inv = jnp.float32(1.0 / math.sqrt(D))

    mesh = plsc.VectorSubcoreMesh(core_axis_name="c", subcore_axis_name="s")

    def body(q_hbm, k_hbm, v_hbm, src_hbm, dst_hbm, zn_hbm, zd_hbm,
             num_out, den_out,
             src_v, dst_v, q_rows, k_rows, v_rows, pbuf, den_tile,
             num_acc, sem0, sem1, sem2):
        cid = lax.axis_index("c")
        sid = lax.axis_index("s")
        wid = sid * _NSC + cid
        r0 = sid * RT
        offs = [(o, min(_C, RT - o)) for o in range(0, RT, _C)]

        pltpu.sync_copy(zn_hbm.at[pl.ds(0, _C)], v_rows)
        for o, sz in offs:
            pltpu.sync_copy(v_rows.at[pl.ds(0, sz)], num_acc.at[pl.ds(r0 + o, sz)])
        pltpu.sync_copy(zd_hbm, den_tile)
        plsc.subcore_barrier()

        base0 = wid * EPW

        def chunk(t, carry):
            eb = pl.multiple_of(base0 + t * _C, _C)
            pltpu.sync_copy(src_hbm.at[pl.ds(eb, _C)], src_v)
            pltpu.sync_copy(dst_hbm.at[pl.ds(eb, _C)], dst_v)
            cp_q = pltpu.async_copy(q_hbm.at[dst_v], q_rows, sem0)
            cp_k = pltpu.async_copy(k_hbm.at[src_v], k_rows, sem1)
            cp_v = pltpu.async_copy(v_hbm.at[src_v], v_rows, sem2)
            cp_q.wait()
            cp_k.wait()
            cp_v.wait()

            def group(g, c2):
                e0 = g * _LANES
                for j in range(_LANES):
                    e = e0 + j
                    acc = q_rows[e, pl.ds(0, _LANES)] * k_rows[e, pl.ds(0, _LANES)]
                    for d in range(1, ND):
                        acc = acc + (q_rows[e, pl.ds(d * _LANES, _LANES)]
                                     * k_rows[e, pl.ds(d * _LANES, _LANES)])
                    pbuf[j, :] = acc
                iota = lax.iota(jnp.int32, _LANES)
                alpha = plsc.load_gather(pbuf, [iota, jnp.zeros((_LANES,), jnp.int32)])
                for l in range(1, _LANES):
                    alpha = alpha + plsc.load_gather(
                        pbuf, [iota, jnp.full((_LANES,), l, jnp.int32)])
                w16 = jnp.exp(alpha * inv)
                dst16 = dst_v[pl.ds(e0, _LANES)]
                plsc.addupdate_scatter(den_tile, [dst16], w16)
                for j in range(_LANES):
                    e = e0 + j
                    w = jnp.full((_LANES,), w16[j], jnp.float32)
                    for d in range(ND):
                        sl = pl.ds(d * _LANES, _LANES)
                        v_rows[e, sl] = v_rows[e, sl] * w
                return c2

            lax.fori_loop(0, _C // _LANES, group, 0)
            pltpu.sync_copy(v_rows, num_acc.at[dst_v], add=True)
            return carry

        lax.fori_loop(0, T, chunk, 0)
        plsc.subcore_barrier()

        for o, sz in offs:
            pltpu.sync_copy(num_acc.at[pl.ds(r0 + o, sz)], v_rows.at[pl.ds(0, sz)])
            pltpu.sync_copy(v_rows.at[pl.ds(0, sz)], num_out.at[cid, pl.ds(r0 + o, sz)])
        pltpu.sync_copy(den_tile, den_out.at[cid, sid])

    call = pl.kernel(
        body,
        compiler_params=pltpu.CompilerParams(needs_layout_passes=False),
        out_type=[
            jax.ShapeDtypeStruct((_NSC, NP, D), jnp.float32),
            jax.ShapeDtypeStruct((_NSC, _NTEC, NP), jnp.float32),
        ],
        mesh=mesh,
        scratch_types=[
            pltpu.VMEM((_C,), jnp.int32),
            pltpu.VMEM((_C,), jnp.int32),
            pltpu.VMEM((_C, D), jnp.float32),
            pltpu.VMEM((_C, D), jnp.float32),
            pltpu.VMEM((_C, D), jnp.float32),
            pltpu.VMEM((_LANES, _LANES), jnp.float32),
            pltpu.VMEM((NP,), jnp.float32),
            pltpu.VMEM_SHARED((NP, D), jnp.float32),
            pltpu.SemaphoreType.DMA,
            pltpu.SemaphoreType.DMA,
            pltpu.SemaphoreType.DMA,
        ],
    )
    return call(q, k, v, srcp, dstp, zeros_n, zeros_d)


def kernel(x, edge_index, Wq1, bq1, Wk1, bk1, Wv1, bv1, Ws1, bs1,
           Wq2, bq2, Wk2, bk2, Wv2, bv2, Ws2, bs2):
    N, D = x.shape
    E = edge_index.shape[1]
    NP = -(-(N + 1) // 128) * 128
    assert (NP // _NTEC) % 8 == 0 and (NP // 4) % 8 == 0

    T = -(-E // (_NW * _C))
    EP = _NW * T * _C

    src = edge_index[0]
    dst = edge_index[1]
    pad = jnp.full((EP - E,), N, jnp.int32)
    srcp = jnp.concatenate([src, pad])
    dstp = jnp.concatenate([dst, pad])

    xp = jnp.concatenate([x, jnp.zeros((NP - N, D), jnp.float32)])
    zeros_n = jnp.zeros((NP, D), jnp.float32)
    zeros_d = jnp.zeros((NP,), jnp.float32)

    q1, k1, v1, s1 = _proj4(xp, Wq1, bq1, Wk1, bk1, Wv1, bv1, Ws1, bs1)
    num1, den1 = _edge_pass(q1, k1, v1, srcp, dstp, zeros_n, zeros_d)
    den1t = den1.reshape(_NW, NP).T
    q2, k2, v2, s2 = _combine_proj4(num1, den1t, s1,
                                    Wq2, bq2, Wk2, bk2, Wv2, bv2, Ws2, bs2)
    num2, den2 = _edge_pass(q2, k2, v2, srcp, dstp, zeros_n, zeros_d)
    den2t = den2.reshape(_NW, NP).T
    out = _combine_final(num2, den2t, s2)
    return out[:N]


# speedup vs baseline: 11.8547x; 1.3428x over previous
import math

import jax
import jax.numpy as jnp
from jax import lax
from jax.experimental import pallas as pl
from jax.experimental.pallas import tpu as pltpu
from jax.experimental.pallas import tpu_sc as plsc

_C = 48
_NSC = 2
_NTEC = 16
_NW = _NSC * _NTEC
_LANES = 16


def _proj4(xp, Wq, bq, Wk, bk, Wv, bv, Ws, bs):
    NP, D = xp.shape
    R = NP // 4
    bspec_x = pl.BlockSpec((R, D), lambda i: (i, 0))
    bspec_w = pl.BlockSpec((D, D), lambda i: (0, 0))
    bspec_b = pl.BlockSpec((1, D), lambda i: (0, 0))

    def body(x_ref, wq, bq_r, wk, bk_r, wv, bv_r, ws, bs_r, qo, ko, vo, so):
        xb = x_ref[...]
        qo[...] = jnp.dot(xb, wq[...], preferred_element_type=jnp.float32) + bq_r[...]
        ko[...] = jnp.dot(xb, wk[...], preferred_element_type=jnp.float32) + bk_r[...]
        vo[...] = jnp.dot(xb, wv[...], preferred_element_type=jnp.float32) + bv_r[...]
        so[...] = jnp.dot(xb, ws[...], preferred_element_type=jnp.float32) + bs_r[...]

    outs = [jax.ShapeDtypeStruct((NP, D), jnp.float32)] * 4
    return pl.pallas_call(
        body,
        grid=(4,),
        in_specs=[bspec_x] + [bspec_w, bspec_b] * 4,
        out_specs=[bspec_x] * 4,
        out_shape=outs,
    )(xp, Wq, bq.reshape(1, D), Wk, bk.reshape(1, D),
      Wv, bv.reshape(1, D), Ws, bs.reshape(1, D))


def _combine_proj4(num_p, den_p, skip, Wq, bq, Wk, bk, Wv, bv, Ws, bs):
    _, NP, D = num_p.shape
    R = NP // 4
    bspec_n = pl.BlockSpec((_NSC, R, D), lambda i: (0, i, 0))
    bspec_d = pl.BlockSpec((R, _NW), lambda i: (i, 0))
    bspec_x = pl.BlockSpec((R, D), lambda i: (i, 0))
    bspec_w = pl.BlockSpec((D, D), lambda i: (0, 0))
    bspec_b = pl.BlockSpec((1, D), lambda i: (0, 0))

    def body(num_ref, den_ref, skip_ref, wq, bq_r, wk, bk_r, wv, bv_r, ws, bs_r,
             qo, ko, vo, so):
        num = num_ref[0] + num_ref[1]
        den = jnp.sum(den_ref[...], axis=1, keepdims=True)
        h = num / (den + 1e-16) + skip_ref[...]
        h = jnp.where(h >= 0, h, 0.1 * h)
        qo[...] = jnp.dot(h, wq[...], preferred_element_type=jnp.float32) + bq_r[...]
        ko[...] = jnp.dot(h, wk[...], preferred_element_type=jnp.float32) + bk_r[...]
        vo[...] = jnp.dot(h, wv[...], preferred_element_type=jnp.float32) + bv_r[...]
        so[...] = jnp.dot(h, ws[...], preferred_element_type=jnp.float32) + bs_r[...]

    outs = [jax.ShapeDtypeStruct((NP, D), jnp.float32)] * 4
    return pl.pallas_call(
        body,
        grid=(4,),
        in_specs=[bspec_n, bspec_d, bspec_x] + [bspec_w, bspec_b] * 4,
        out_specs=[bspec_x] * 4,
        out_shape=outs,
    )(num_p, den_p, skip, Wq, bq.reshape(1, D), Wk, bk.reshape(1, D),
      Wv, bv.reshape(1, D), Ws, bs.reshape(1, D))


def _combine_final(num_p, den_p, skip):
    _, NP, D = num_p.shape
    R = NP // 4
    bspec_n = pl.BlockSpec((_NSC, R, D), lambda i: (0, i, 0))
    bspec_d = pl.BlockSpec((R, _NW), lambda i: (i, 0))
    bspec_x = pl.BlockSpec((R, D), lambda i: (i, 0))

    def body(num_ref, den_ref, skip_ref, out_ref):
        num = num_ref[0] + num_ref[1]
        den = jnp.sum(den_ref[...], axis=1, keepdims=True)
        out_ref[...] = num / (den + 1e-16) + skip_ref[...]

    return pl.pallas_call(
        body,
        grid=(4,),
        in_specs=[bspec_n, bspec_d, bspec_x],
        out_specs=bspec_x,
        out_shape=jax.ShapeDtypeStruct((NP, D), jnp.float32),
    )(num_p, den_p, skip)


def _edge_pass(q, k, v, srcp, dstp, zeros_n, zeros_d):
    NP, D = q.shape
    EP = srcp.shape[0] - 2 * _C
    EPW = EP // _NW
    T = EPW // _C
    RT = NP // _NTEC
    ND = D // _LANES
    inv = jnp.float32(1.0 / math.sqrt(D))

    mesh = plsc.VectorSubcoreMesh(core_axis_name="c", subcore_axis_name="s")

    def body(q_hbm, k_hbm, v_hbm, src_hbm, dst_hbm, zn_hbm, zd_hbm,
             num_out, den_out,
             src_b, dst_b, q_b, k_b, v_rows, wbuf, pbuf, den_tile,
             num_acc, semq, semk, semv, semis, semid):
        cid = lax.axis_index("c")
        sid = lax.axis_index("s")
        wid = sid * _NSC + cid
        r0 = sid * RT
        offs = [(o, min(_C, RT - o)) for o in range(0, RT, _C)]

        pltpu.sync_copy(zn_hbm.at[pl.ds(0, _C)], v_rows)
        for o, sz in offs:
            pltpu.sync_copy(v_rows.at[pl.ds(0, sz)], num_acc.at[pl.ds(r0 + o, sz)])
        pltpu.sync_copy(zd_hbm, den_tile)
        plsc.subcore_barrier()

        base0 = wid * EPW

        def eb(u):
            return pl.multiple_of(base0 + u * _C, 8)

        def compute_w(qr, kr):
            def groupA(g, c2):
                e0 = g * _LANES
                for j in range(_LANES):
                    e = e0 + j
                    acc = qr[e, pl.ds(0, _LANES)] * kr[e, pl.ds(0, _LANES)]
                    for d in range(1, ND):
                        acc = acc + (qr[e, pl.ds(d * _LANES, _LANES)]
                                     * kr[e, pl.ds(d * _LANES, _LANES)])
                    pbuf[j, :] = acc
                iota = lax.iota(jnp.int32, _LANES)
                alpha = plsc.load_gather(pbuf, [iota, jnp.zeros((_LANES,), jnp.int32)])
                for l in range(1, _LANES):
                    alpha = alpha + plsc.load_gather(
                        pbuf, [iota, jnp.full((_LANES,), l, jnp.int32)])
                w16 = jnp.exp(alpha * inv)
                wbuf[pl.ds(e0, _LANES)] = w16
                return c2
            lax.fori_loop(0, _C // _LANES, groupA, 0)

        def scale_v(dst_ref):
            def groupB(g, c2):
                e0 = g * _LANES
                w16 = wbuf[pl.ds(e0, _LANES)]
                dst16 = dst_ref[pl.ds(e0, _LANES)]
                plsc.addupdate_scatter(den_tile, [dst16], w16)
                for j in range(_LANES):
                    e = e0 + j
                    w = jnp.full((_LANES,), w16[j], jnp.float32)
                    for d in range(ND):
                        sl = pl.ds(d * _LANES, _LANES)
                        v_rows[e, sl] = v_rows[e, sl] * w
                return c2
            lax.fori_loop(0, _C // _LANES, groupB, 0)

        pltpu.sync_copy(src_hbm.at[pl.ds(eb(0), _C)], src_b.at[0])
        pltpu.sync_copy(dst_hbm.at[pl.ds(eb(0), _C)], dst_b.at[0])
        pltpu.async_copy(q_hbm.at[dst_b.at[0]], q_b.at[0], semq[0])
        pltpu.async_copy(k_hbm.at[src_b.at[0]], k_b.at[0], semk[0])
        pltpu.async_copy(src_hbm.at[pl.ds(eb(1), _C)], src_b.at[1], semis[1])
        pltpu.async_copy(dst_hbm.at[pl.ds(eb(1), _C)], dst_b.at[1], semid[1])

        def phase(t, cs, ns):
            pltpu.make_async_copy(src_hbm.at[pl.ds(eb(t + 1), _C)],
                                  src_b.at[ns], semis[ns]).wait()
            pltpu.make_async_copy(dst_hbm.at[pl.ds(eb(t + 1), _C)],
                                  dst_b.at[ns], semid[ns]).wait()
            pltpu.async_copy(q_hbm.at[dst_b.at[ns]], q_b.at[ns], semq[ns])
            pltpu.async_copy(k_hbm.at[src_b.at[ns]], k_b.at[ns], semk[ns])
            pltpu.async_copy(v_hbm.at[src_b.at[cs]], v_rows, semv)
            pltpu.make_async_copy(q_hbm.at[dst_b.at[cs]], q_b.at[cs], semq[cs]).wait()
            pltpu.make_async_copy(k_hbm.at[src_b.at[cs]], k_b.at[cs], semk[cs]).wait()
            compute_w(q_b.at[cs], k_b.at[cs])
            pltpu.make_async_copy(v_hbm.at[src_b.at[cs]], v_rows, semv).wait()
            scale_v(dst_b.at[cs])
            pltpu.sync_copy(v_rows, num_acc.at[dst_b.at[cs]], add=True)
            pltpu.async_copy(src_hbm.at[pl.ds(eb(t + 2), _C)], src_b.at[cs], semis[cs])
            pltpu.async_copy(dst_hbm.at[pl.ds(eb(t + 2), _C)], dst_b.at[cs], semid[cs])

        def pair(i, carry):
            phase(2 * i, 0, 1)
            phase(2 * i + 1, 1, 0)
            return carry

        lax.fori_loop(0, T // 2, pair, 0)

        pltpu.make_async_copy(q_hbm.at[dst_b.at[0]], q_b.at[0], semq[0]).wait()
        pltpu.make_async_copy(k_hbm.at[src_b.at[0]], k_b.at[0], semk[0]).wait()
        pltpu.make_async_copy(src_hbm.at[pl.ds(0, _C)], src_b.at[1], semis[1]).wait()
        pltpu.make_async_copy(dst_hbm.at[pl.ds(0, _C)], dst_b.at[1], semid[1]).wait()
        plsc.subcore_barrier()

        for o, sz in offs:
            pltpu.sync_copy(num_acc.at[pl.ds(r0 + o, sz)], v_rows.at[pl.ds(0, sz)])
            pltpu.sync_copy(v_rows.at[pl.ds(0, sz)], num_out.at[cid, pl.ds(r0 + o, sz)])
        pltpu.sync_copy(den_tile, den_out.at[cid, sid])

    call = pl.kernel(
        body,
        compiler_params=pltpu.CompilerParams(needs_layout_passes=False),
        out_type=[
            jax.ShapeDtypeStruct((_NSC, NP, D), jnp.float32),
            jax.ShapeDtypeStruct((_NSC, _NTEC, NP), jnp.float32),
        ],
        mesh=mesh,
        scratch_types=[
            pltpu.VMEM((2, _C), jnp.int32),
            pltpu.VMEM((2, _C), jnp.int32),
            pltpu.VMEM((2, _C, D), jnp.float32),
            pltpu.VMEM((2, _C, D), jnp.float32),
            pltpu.VMEM((_C, D), jnp.float32),
            pltpu.VMEM((_C,), jnp.float32),
            pltpu.VMEM((_LANES, _LANES), jnp.float32),
            pltpu.VMEM((NP,), jnp.float32),
            pltpu.VMEM_SHARED((NP, D), jnp.float32),
            [pltpu.SemaphoreType.DMA] * 2,
            [pltpu.SemaphoreType.DMA] * 2,
            pltpu.SemaphoreType.DMA,
            [pltpu.SemaphoreType.DMA] * 2,
            [pltpu.SemaphoreType.DMA] * 2,
        ],
    )
    return call(q, k, v, srcp, dstp, zeros_n, zeros_d)


def kernel(x, edge_index, Wq1, bq1, Wk1, bk1, Wv1, bv1, Ws1, bs1,
           Wq2, bq2, Wk2, bk2, Wv2, bv2, Ws2, bs2):
    N, D = x.shape
    E = edge_index.shape[1]
    NP = -(-(N + 1) // 128) * 128
    assert (NP // _NTEC) % 8 == 0 and (NP // 4) % 8 == 0

    T = -(-E // (_NW * _C))
    if T % 2:
        T += 1
    EP = _NW * T * _C

    src = edge_index[0]
    dst = edge_index[1]
    pad = jnp.full((EP - E + 2 * _C,), N, jnp.int32)
    srcp = jnp.concatenate([src, pad])
    dstp = jnp.concatenate([dst, pad])

    xp = jnp.concatenate([x, jnp.zeros((NP - N, D), jnp.float32)])
    zeros_n = jnp.zeros((NP, D), jnp.float32)
    zeros_d = jnp.zeros((NP,), jnp.float32)

    q1, k1, v1, s1 = _proj4(xp, Wq1, bq1, Wk1, bk1, Wv1, bv1, Ws1, bs1)
    num1, den1 = _edge_pass(q1, k1, v1, srcp, dstp, zeros_n, zeros_d)
    den1t = den1.reshape(_NW, NP).T
    q2, k2, v2, s2 = _combine_proj4(num1, den1t, s1,
                                    Wq2, bq2, Wk2, bk2, Wv2, bv2, Ws2, bs2)
    num2, den2 = _edge_pass(q2, k2, v2, srcp, dstp, zeros_n, zeros_d)
    den2t = den2.reshape(_NW, NP).T
    out = _combine_final(num2, den2t, s2)
    return out[:N]


# speedup vs baseline: 12.7659x; 1.0769x over previous
import math

import jax
import jax.numpy as jnp
from jax import lax
from jax.experimental import pallas as pl
from jax.experimental.pallas import tpu as pltpu
from jax.experimental.pallas import tpu_sc as plsc

_C = 48
_NSC = 2
_NTEC = 16
_NW = _NSC * _NTEC
_LANES = 16


def _proj4(xp, Wq, bq, Wk, bk, Wv, bv, Ws, bs):
    NP, D = xp.shape
    R = NP // 4
    bspec_x = pl.BlockSpec((R, D), lambda i: (i, 0))
    bspec_w = pl.BlockSpec((D, D), lambda i: (0, 0))
    bspec_b = pl.BlockSpec((1, D), lambda i: (0, 0))

    def body(x_ref, wq, bq_r, wk, bk_r, wv, bv_r, ws, bs_r, qo, ko, vo, so):
        xb = x_ref[...]
        qo[...] = jnp.dot(xb, wq[...], preferred_element_type=jnp.float32) + bq_r[...]
        ko[...] = jnp.dot(xb, wk[...], preferred_element_type=jnp.float32) + bk_r[...]
        vo[...] = jnp.dot(xb, wv[...], preferred_element_type=jnp.float32) + bv_r[...]
        so[...] = jnp.dot(xb, ws[...], preferred_element_type=jnp.float32) + bs_r[...]

    outs = [jax.ShapeDtypeStruct((NP, D), jnp.float32)] * 4
    return pl.pallas_call(
        body,
        grid=(4,),
        in_specs=[bspec_x] + [bspec_w, bspec_b] * 4,
        out_specs=[bspec_x] * 4,
        out_shape=outs,
    )(xp, Wq, bq.reshape(1, D), Wk, bk.reshape(1, D),
      Wv, bv.reshape(1, D), Ws, bs.reshape(1, D))


def _combine_proj4(num_p, den_p, skip, Wq, bq, Wk, bk, Wv, bv, Ws, bs):
    _, NP, D = num_p.shape
    R = NP // 4
    bspec_n = pl.BlockSpec((_NSC, R, D), lambda i: (0, i, 0))
    bspec_d = pl.BlockSpec((R, _NW), lambda i: (i, 0))
    bspec_x = pl.BlockSpec((R, D), lambda i: (i, 0))
    bspec_w = pl.BlockSpec((D, D), lambda i: (0, 0))
    bspec_b = pl.BlockSpec((1, D), lambda i: (0, 0))

    def body(num_ref, den_ref, skip_ref, wq, bq_r, wk, bk_r, wv, bv_r, ws, bs_r,
             qo, ko, vo, so):
        num = num_ref[0] + num_ref[1]
        den = jnp.sum(den_ref[...], axis=1, keepdims=True)
        h = num / (den + 1e-16) + skip_ref[...]
        h = jnp.where(h >= 0, h, 0.1 * h)
        qo[...] = jnp.dot(h, wq[...], preferred_element_type=jnp.float32) + bq_r[...]
        ko[...] = jnp.dot(h, wk[...], preferred_element_type=jnp.float32) + bk_r[...]
        vo[...] = jnp.dot(h, wv[...], preferred_element_type=jnp.float32) + bv_r[...]
        so[...] = jnp.dot(h, ws[...], preferred_element_type=jnp.float32) + bs_r[...]

    outs = [jax.ShapeDtypeStruct((NP, D), jnp.float32)] * 4
    return pl.pallas_call(
        body,
        grid=(4,),
        in_specs=[bspec_n, bspec_d, bspec_x] + [bspec_w, bspec_b] * 4,
        out_specs=[bspec_x] * 4,
        out_shape=outs,
    )(num_p, den_p, skip, Wq, bq.reshape(1, D), Wk, bk.reshape(1, D),
      Wv, bv.reshape(1, D), Ws, bs.reshape(1, D))


def _combine_final(num_p, den_p, skip):
    _, NP, D = num_p.shape
    R = NP // 4
    bspec_n = pl.BlockSpec((_NSC, R, D), lambda i: (0, i, 0))
    bspec_d = pl.BlockSpec((R, _NW), lambda i: (i, 0))
    bspec_x = pl.BlockSpec((R, D), lambda i: (i, 0))

    def body(num_ref, den_ref, skip_ref, out_ref):
        num = num_ref[0] + num_ref[1]
        den = jnp.sum(den_ref[...], axis=1, keepdims=True)
        out_ref[...] = num / (den + 1e-16) + skip_ref[...]

    return pl.pallas_call(
        body,
        grid=(4,),
        in_specs=[bspec_n, bspec_d, bspec_x],
        out_specs=bspec_x,
        out_shape=jax.ShapeDtypeStruct((NP, D), jnp.float32),
    )(num_p, den_p, skip)


def _edge_pass(q, k, v, srcp, dstp, zeros_n, zeros_d):
    NP, D = q.shape
    EP = srcp.shape[0] - 2 * _C
    EPW = EP // _NW
    T = EPW // _C
    RT = NP // _NTEC
    ND = D // _LANES
    inv = jnp.float32(1.0 / math.sqrt(D))

    mesh = plsc.VectorSubcoreMesh(core_axis_name="c", subcore_axis_name="s")

    def body(q_hbm, k_hbm, v_hbm, src_hbm, dst_hbm, zn_hbm, zd_hbm,
             num_out, den_out,
             src_b, dst_b, q_b, k_b, v_rows, wbuf, pbuf, den_tile,
             num_acc, semq, semk, semv, semis, semid, semsc):
        cid = lax.axis_index("c")
        sid = lax.axis_index("s")
        wid = sid * _NSC + cid
        r0 = sid * RT
        offs = [(o, min(_C, RT - o)) for o in range(0, RT, _C)]

        pltpu.sync_copy(zn_hbm.at[pl.ds(0, _C)], v_rows)
        for o, sz in offs:
            pltpu.sync_copy(v_rows.at[pl.ds(0, sz)], num_acc.at[pl.ds(r0 + o, sz)])
        pltpu.sync_copy(zd_hbm, den_tile)
        plsc.subcore_barrier()

        base0 = wid * EPW

        def eb(u):
            return pl.multiple_of(base0 + u * _C, 8)

        def compute_w(qr, kr):
            def groupA(g, c2):
                e0 = g * _LANES
                for j in range(_LANES):
                    e = e0 + j
                    acc = qr[e, pl.ds(0, _LANES)] * kr[e, pl.ds(0, _LANES)]
                    for d in range(1, ND):
                        acc = acc + (qr[e, pl.ds(d * _LANES, _LANES)]
                                     * kr[e, pl.ds(d * _LANES, _LANES)])
                    pbuf[j, :] = acc
                iota = lax.iota(jnp.int32, _LANES)
                alpha = plsc.load_gather(pbuf, [iota, jnp.zeros((_LANES,), jnp.int32)])
                for l in range(1, _LANES):
                    alpha = alpha + plsc.load_gather(
                        pbuf, [iota, jnp.full((_LANES,), l, jnp.int32)])
                w16 = jnp.exp(alpha * inv)
                wbuf[pl.ds(e0, _LANES)] = w16
                return c2
            lax.fori_loop(0, _C // _LANES, groupA, 0)

        def scale_v(dst_ref):
            def groupB(g, c2):
                e0 = g * _LANES
                w16 = wbuf[pl.ds(e0, _LANES)]
                dst16 = dst_ref[pl.ds(e0, _LANES)]
                plsc.addupdate_scatter(den_tile, [dst16], w16)
                for j in range(_LANES):
                    e = e0 + j
                    w = jnp.full((_LANES,), w16[j], jnp.float32)
                    for d in range(ND):
                        sl = pl.ds(d * _LANES, _LANES)
                        v_rows[e, sl] = v_rows[e, sl] * w
                return c2
            lax.fori_loop(0, _C // _LANES, groupB, 0)

        pltpu.sync_copy(src_hbm.at[pl.ds(eb(0), _C)], src_b.at[0])
        pltpu.sync_copy(dst_hbm.at[pl.ds(eb(0), _C)], dst_b.at[0])
        pltpu.async_copy(q_hbm.at[dst_b.at[0]], q_b.at[0], semq[0])
        pltpu.async_copy(k_hbm.at[src_b.at[0]], k_b.at[0], semk[0])
        pltpu.async_copy(src_hbm.at[pl.ds(eb(1), _C)], src_b.at[1], semis[1])
        pltpu.async_copy(dst_hbm.at[pl.ds(eb(1), _C)], dst_b.at[1], semid[1])
        pltpu.async_copy(v_rows, num_acc.at[dst_b.at[0]], semsc, add=True)

        def phase(t, cs, ns, ds_cur, ds_nxt, ds_nxt2):
            pltpu.make_async_copy(src_hbm.at[pl.ds(eb(t + 1), _C)],
                                  src_b.at[ns], semis[ns]).wait()
            pltpu.make_async_copy(dst_hbm.at[pl.ds(eb(t + 1), _C)],
                                  dst_b.at[ds_nxt], semid[ds_nxt]).wait()
            pltpu.async_copy(q_hbm.at[dst_b.at[ds_nxt]], q_b.at[ns], semq[ns])
            pltpu.async_copy(k_hbm.at[src_b.at[ns]], k_b.at[ns], semk[ns])
            pltpu.make_async_copy(v_rows, num_acc.at[dst_b.at[ds_cur]],
                                  semsc).wait()
            pltpu.async_copy(v_hbm.at[src_b.at[cs]], v_rows, semv)
            pltpu.make_async_copy(q_hbm.at[dst_b.at[ds_cur]], q_b.at[cs], semq[cs]).wait()
            pltpu.make_async_copy(k_hbm.at[src_b.at[cs]], k_b.at[cs], semk[cs]).wait()
            compute_w(q_b.at[cs], k_b.at[cs])
            pltpu.make_async_copy(v_hbm.at[src_b.at[cs]], v_rows, semv).wait()
            scale_v(dst_b.at[ds_cur])
            pltpu.async_copy(v_rows, num_acc.at[dst_b.at[ds_cur]], semsc, add=True)
            pltpu.async_copy(src_hbm.at[pl.ds(eb(t + 2), _C)], src_b.at[cs], semis[cs])
            pltpu.async_copy(dst_hbm.at[pl.ds(eb(t + 2), _C)],
                             dst_b.at[ds_nxt2], semid[ds_nxt2])

        def six(i, carry):
            t0 = 6 * i
            for p in range(6):
                phase(t0 + p, p % 2, (p + 1) % 2, p % 3, (p + 1) % 3, (p + 2) % 3)
            return carry

        lax.fori_loop(0, T // 6, six, 0)

        pltpu.make_async_copy(v_rows, num_acc.at[dst_b.at[(T - 1) % 3]],
                              semsc).wait()
        pltpu.make_async_copy(q_hbm.at[dst_b.at[T % 3]], q_b.at[T % 2],
                              semq[T % 2]).wait()
        pltpu.make_async_copy(k_hbm.at[src_b.at[T % 2]], k_b.at[T % 2],
                              semk[T % 2]).wait()
        pltpu.make_async_copy(src_hbm.at[pl.ds(0, _C)], src_b.at[(T - 1) % 2],
                              semis[(T - 1) % 2]).wait()
        pltpu.make_async_copy(dst_hbm.at[pl.ds(0, _C)], dst_b.at[(T + 1) % 3],
                              semid[(T + 1) % 3]).wait()
        plsc.subcore_barrier()

        for o, sz in offs:
            pltpu.sync_copy(num_acc.at[pl.ds(r0 + o, sz)], v_rows.at[pl.ds(0, sz)])
            pltpu.sync_copy(v_rows.at[pl.ds(0, sz)], num_out.at[cid, pl.ds(r0 + o, sz)])
        pltpu.sync_copy(den_tile, den_out.at[cid, sid])

    call = pl.kernel(
        body,
        compiler_params=pltpu.CompilerParams(needs_layout_passes=False),
        out_type=[
            jax.ShapeDtypeStruct((_NSC, NP, D), jnp.float32),
            jax.ShapeDtypeStruct((_NSC, _NTEC, NP), jnp.float32),
        ],
        mesh=mesh,
        scratch_types=[
            pltpu.VMEM((2, _C), jnp.int32),
            pltpu.VMEM((3, _C), jnp.int32),
            pltpu.VMEM((2, _C, D), jnp.float32),
            pltpu.VMEM((2, _C, D), jnp.float32),
            pltpu.VMEM((_C, D), jnp.float32),
            pltpu.VMEM((_C,), jnp.float32),
            pltpu.VMEM((_LANES, _LANES), jnp.float32),
            pltpu.VMEM((NP,), jnp.float32),
            pltpu.VMEM_SHARED((NP, D), jnp.float32),
            [pltpu.SemaphoreType.DMA] * 2,
            [pltpu.SemaphoreType.DMA] * 2,
            pltpu.SemaphoreType.DMA,
            [pltpu.SemaphoreType.DMA] * 2,
            [pltpu.SemaphoreType.DMA] * 3,
            pltpu.SemaphoreType.DMA,
        ],
    )
    return call(q, k, v, srcp, dstp, zeros_n, zeros_d)


def kernel(x, edge_index, Wq1, bq1, Wk1, bk1, Wv1, bv1, Ws1, bs1,
           Wq2, bq2, Wk2, bk2, Wv2, bv2, Ws2, bs2):
    N, D = x.shape
    E = edge_index.shape[1]
    NP = -(-(N + 1) // 128) * 128
    assert (NP // _NTEC) % 8 == 0 and (NP // 4) % 8 == 0

    T = -(-E // (_NW * _C))
    T += (-T) % 6
    EP = _NW * T * _C

    src = edge_index[0]
    dst = edge_index[1]
    pad = jnp.full((EP - E + 2 * _C,), N, jnp.int32)
    srcp = jnp.concatenate([src, pad])
    dstp = jnp.concatenate([dst, pad])

    xp = jnp.concatenate([x, jnp.zeros((NP - N, D), jnp.float32)])
    zeros_n = jnp.zeros((NP, D), jnp.float32)
    zeros_d = jnp.zeros((NP,), jnp.float32)

    q1, k1, v1, s1 = _proj4(xp, Wq1, bq1, Wk1, bk1, Wv1, bv1, Ws1, bs1)
    num1, den1 = _edge_pass(q1, k1, v1, srcp, dstp, zeros_n, zeros_d)
    den1t = den1.reshape(_NW, NP).T
    q2, k2, v2, s2 = _combine_proj4(num1, den1t, s1,
                                    Wq2, bq2, Wk2, bk2, Wv2, bv2, Ws2, bs2)
    num2, den2 = _edge_pass(q2, k2, v2, srcp, dstp, zeros_n, zeros_d)
    den2t = den2.reshape(_NW, NP).T
    out = _combine_final(num2, den2t, s2)
    return out[:N]
